# baseline (speedup 1.0000x reference)
import functools

import jax
import jax.numpy as jnp
from jax import lax
from jax.experimental import pallas as pl
from jax.experimental.pallas import tpu as pltpu
from jax.experimental.pallas import tpu_sc as plsc

BATCH = 16384
HIST = 50
EMBED_DIM = 64
VOCAB = 1000

K = 4


def _build():
    info = plsc.get_sparse_core_info()
    nw = info.num_cores * info.num_subcores
    b_per_w = BATCH // nw
    n_chunks = b_per_w // K
    assert b_per_w % K == 0 and (n_chunks - 2) % 2 == 0 and n_chunks >= 4

    mesh = plsc.VectorSubcoreMesh(core_axis_name="c", subcore_axis_name="s")

    @functools.partial(
        pl.kernel,
        mesh=mesh,
        out_type=jax.ShapeDtypeStruct((BATCH, HIST, EMBED_DIM), jnp.float32),
        scratch_types=[
            pltpu.VMEM((2, K, HIST), jnp.int32),
            pltpu.VMEM((2, K, HIST, EMBED_DIM), jnp.float32),
            [pltpu.SemaphoreType.DMA] * 2,
            [pltpu.SemaphoreType.DMA] * 2,
            [pltpu.SemaphoreType.DMA] * 2,
        ],
        compiler_params=pltpu.CompilerParams(use_tc_tiling_on_sc=False),
    )
    def kern(idx_hbm, table_hbm, out_hbm, idx_v, rows_v, isems, gsems, osems):
        wid = lax.axis_index("s") * info.num_cores + lax.axis_index("c")
        b0 = wid * b_per_w

        def idx_copy(chunk, s):
            pltpu.async_copy(
                idx_hbm.at[pl.ds(b0 + chunk * K, K)], idx_v.at[s], isems[s]
            )

        def fire_gathers(s):
            for j in range(K):
                pltpu.async_copy(
                    table_hbm.at[idx_v.at[s].at[j]], rows_v.at[s].at[j], gsems[s]
                )

        def out_write(chunk, s):
            pltpu.async_copy(
                rows_v.at[s], out_hbm.at[pl.ds(b0 + chunk * K, K)], osems[s]
            )

        def drain_idx(s):
            pltpu.make_async_copy(idx_hbm.at[pl.ds(0, K)], idx_v.at[s], isems[s]).wait()

        def drain_gathers(s):
            for j in range(K):
                pltpu.make_async_copy(
                    table_hbm.at[pl.ds(0, HIST)], rows_v.at[s].at[j], gsems[s]
                ).wait()

        def drain_write(s):
            pltpu.make_async_copy(
                rows_v.at[s], out_hbm.at[pl.ds(0, K)], osems[s]
            ).wait()

        def step(i, s, first):
            drain_gathers(s)
            out_write(i, s)
            s1 = 1 - s
            drain_idx(s1)
            if not first:
                drain_write(s1)
            fire_gathers(s1)
            idx_copy(jnp.minimum(i + 2, n_chunks - 1), s)

        idx_copy(0, 0)
        drain_idx(0)
        fire_gathers(0)
        idx_copy(1, 1)

        step(0, 0, first=True)

        def round_body(r, carry):
            step(2 * r + 1, 1, first=False)
            step(2 * r + 2, 0, first=False)
            return carry

        lax.fori_loop(0, (n_chunks - 2) // 2, round_body, 0)

        drain_gathers(1)
        out_write(n_chunks - 1, 1)
        drain_idx(0)
        drain_write(0)
        drain_write(1)

    return kern


_kern = _build()


REP = 16


def kernel(indices, table):
    table_rep = jnp.tile(table, (REP, 1))
    off = (jnp.arange(BATCH, dtype=jnp.int32) % REP)[:, None] * VOCAB
    idx_off = indices.astype(jnp.int32) + off
    return _kern(idx_off, table_rep)


# speedup vs baseline: 1.0029x; 1.0029x over previous
import functools

import jax
import jax.numpy as jnp
from jax import lax
from jax.experimental import pallas as pl
from jax.experimental.pallas import tpu as pltpu
from jax.experimental.pallas import tpu_sc as plsc

BATCH = 16384
HIST = 50
EMBED_DIM = 64
VOCAB = 1000

K = 4


def _build():
    info = plsc.get_sparse_core_info()
    nw = info.num_cores * info.num_subcores
    b_per_w = BATCH // nw
    n_chunks = b_per_w // K
    assert b_per_w % K == 0 and (n_chunks - 2) % 2 == 0 and n_chunks >= 4

    mesh = plsc.VectorSubcoreMesh(core_axis_name="c", subcore_axis_name="s")

    @functools.partial(
        pl.kernel,
        mesh=mesh,
        out_type=jax.ShapeDtypeStruct((BATCH, HIST, EMBED_DIM), jnp.float32),
        scratch_types=[
            pltpu.VMEM((2, K, HIST), jnp.int32),
            pltpu.VMEM((2, K, HIST, EMBED_DIM), jnp.float32),
            [pltpu.SemaphoreType.DMA] * 2,
            [pltpu.SemaphoreType.DMA] * 2,
            [pltpu.SemaphoreType.DMA] * 2,
        ],
        compiler_params=pltpu.CompilerParams(use_tc_tiling_on_sc=False),
    )
    def kern(idx_hbm, table_hbm, out_hbm, idx_v, rows_v, isems, gsems, osems):
        wid = lax.axis_index("s") * info.num_cores + lax.axis_index("c")
        b0 = wid * b_per_w

        def idx_copy(chunk, s):
            pltpu.async_copy(
                idx_hbm.at[pl.ds(b0 + chunk * K, K)], idx_v.at[s], isems[s]
            )

        def fire_gathers(s):
            for j in range(K):
                pltpu.async_copy(
                    table_hbm.at[idx_v.at[s].at[j]], rows_v.at[s].at[j], gsems[s]
                )

        def out_write(chunk, s):
            pltpu.async_copy(
                rows_v.at[s], out_hbm.at[pl.ds(b0 + chunk * K, K)], osems[s]
            )

        def drain_idx(s):
            pltpu.make_async_copy(idx_hbm.at[pl.ds(0, K)], idx_v.at[s], isems[s]).wait()

        def drain_gathers(s):
            for j in range(K):
                pltpu.make_async_copy(
                    table_hbm.at[pl.ds(0, HIST)], rows_v.at[s].at[j], gsems[s]
                ).wait()

        def drain_write(s):
            pltpu.make_async_copy(
                rows_v.at[s], out_hbm.at[pl.ds(0, K)], osems[s]
            ).wait()

        def step(i, s, first):
            drain_gathers(s)
            out_write(i, s)
            s1 = 1 - s
            drain_idx(s1)
            if not first:
                drain_write(s1)
            fire_gathers(s1)
            idx_copy(jnp.minimum(i + 2, n_chunks - 1), s)

        idx_copy(0, 0)
        drain_idx(0)
        fire_gathers(0)
        idx_copy(1, 1)

        step(0, 0, first=True)

        def round_body(r, carry):
            step(2 * r + 1, 1, first=False)
            step(2 * r + 2, 0, first=False)
            return carry

        lax.fori_loop(0, (n_chunks - 2) // 2, round_body, 0)

        drain_gathers(1)
        out_write(n_chunks - 1, 1)
        drain_idx(0)
        drain_write(0)
        drain_write(1)

    return kern


_kern = _build()


REP = 8


def kernel(indices, table):
    table_rep = jnp.tile(table, (REP, 1))
    off = (jnp.arange(BATCH, dtype=jnp.int32) % REP)[:, None] * VOCAB
    idx_off = indices.astype(jnp.int32) + off
    return _kern(idx_off, table_rep)
